# baseline (speedup 1.0000x reference)
import functools

import jax
import jax.numpy as jnp
from jax import lax
from jax.experimental import pallas as pl
from jax.experimental.pallas import tpu as pltpu
from jax.experimental.pallas import tpu_sc as plsc

D = 768

NUM_CORES = 2
NUM_SUBCORES = 16
NW = NUM_CORES * NUM_SUBCORES

SEQ_PAD = 80
IDX_CHUNK_BLKS = 32

TC_G = 16


def _sc_gather_body(ids_hbm, tok_tab, out_hbm, idx_v, rows_v, sem_g, sem_o,
                    *, blocks_per_w):
    wid = lax.axis_index("s") * NUM_CORES + lax.axis_index("c")
    base_row = wid * blocks_per_w * SEQ_PAD
    n_chunks = blocks_per_w // IDX_CHUNK_BLKS

    def gather(i, buf):
        return pltpu.make_async_copy(
            tok_tab.at[idx_v.at[pl.ds(i * SEQ_PAD, SEQ_PAD)]],
            rows_v.at[buf], sem_g)

    def writeout(base, i, buf):
        return pltpu.make_async_copy(
            rows_v.at[buf], out_hbm.at[pl.ds(base + i * SEQ_PAD, SEQ_PAD)],
            sem_o)

    def chunk_body(c, carry):
        chunk_row = base_row + c * IDX_CHUNK_BLKS * SEQ_PAD
        pltpu.sync_copy(ids_hbm.at[pl.ds(chunk_row, IDX_CHUNK_BLKS * SEQ_PAD)],
                        idx_v)
        gather(0, 0).start()

        def body(i, carry2):
            buf = lax.rem(i, 2)
            nbuf = lax.rem(i + 1, 2)

            @pl.when(i >= 1)
            def _():
                writeout(chunk_row, i - 1, nbuf).wait()

            @pl.when(i + 1 < IDX_CHUNK_BLKS)
            def _():
                gather(i + 1, nbuf).start()

            gather(i, buf).wait()
            writeout(chunk_row, i, buf).start()
            return carry2

        lax.fori_loop(0, IDX_CHUNK_BLKS, body, 0)
        writeout(chunk_row, IDX_CHUNK_BLKS - 1,
                 lax.rem(IDX_CHUNK_BLKS - 1, 2)).wait()
        return carry

    lax.fori_loop(0, n_chunks, chunk_body, 0)


def _sc_gather(ids_pad_flat, token_table, n_rows):
    blocks_per_w = (n_rows // SEQ_PAD) // NW
    mesh = plsc.VectorSubcoreMesh(core_axis_name="c", subcore_axis_name="s")
    run = pl.kernel(
        functools.partial(_sc_gather_body, blocks_per_w=blocks_per_w),
        mesh=mesh,
        out_type=jax.ShapeDtypeStruct((n_rows, D), jnp.float32),
        scratch_types=[
            pltpu.VMEM((IDX_CHUNK_BLKS * SEQ_PAD,), jnp.int32),
            pltpu.VMEM((2, SEQ_PAD, D), jnp.float32),
            pltpu.SemaphoreType.DMA,
            pltpu.SemaphoreType.DMA,
        ],
    )
    return run(ids_pad_flat, token_table)


def _tc_body(rows_ref, pid_ref, ptab_ref, out_ref, *, seq):
    g = out_ref.shape[0]
    onehot = (pid_ref[...]
              == lax.broadcasted_iota(jnp.int32, (g * seq, seq), 1)
              ).astype(jnp.float32)
    pos = jax.lax.dot(onehot, ptab_ref[...],
                      preferred_element_type=jnp.float32)
    for b in range(g):
        out_ref[b] = (rows_ref[b * SEQ_PAD:b * SEQ_PAD + seq, :]
                      + pos[b * seq:(b + 1) * seq, :])


def _tc_addpos(tok_rows, pid_flat, position_table, bsz, seq):
    grid = (bsz // TC_G,)
    return pl.pallas_call(
        functools.partial(_tc_body, seq=seq),
        grid=grid,
        in_specs=[
            pl.BlockSpec((TC_G * SEQ_PAD, D), lambda i: (i, 0)),
            pl.BlockSpec((TC_G * seq, 1), lambda i: (i, 0)),
            pl.BlockSpec((seq, D), lambda i: (0, 0)),
        ],
        out_specs=pl.BlockSpec((TC_G, seq, D), lambda i: (i, 0, 0)),
        out_shape=jax.ShapeDtypeStruct((bsz, seq, D), jnp.float32),
    )(tok_rows, pid_flat, position_table)


def kernel(input_ids, position_ids, token_table, position_table):
    bsz, seq = input_ids.shape
    assert seq <= SEQ_PAD
    ids_pad = jnp.pad(input_ids.astype(jnp.int32),
                      ((0, 0), (0, SEQ_PAD - seq))).reshape(bsz * SEQ_PAD)
    tok_rows = _sc_gather(ids_pad, token_table, bsz * SEQ_PAD)
    pid_flat = position_ids.astype(jnp.int32).reshape(bsz * seq, 1)
    return _tc_addpos(tok_rows, pid_flat, position_table, bsz, seq)


# speedup vs baseline: 1.4597x; 1.4597x over previous
import functools

import jax
import jax.numpy as jnp
from jax import lax
from jax.experimental import pallas as pl
from jax.experimental.pallas import tpu as pltpu
from jax.experimental.pallas import tpu_sc as plsc

D = 768

NUM_CORES = 2
NUM_SUBCORES = 16
NW = NUM_CORES * NUM_SUBCORES

SEQ_PAD = 80
IDX_CHUNK_BLKS = 32

TC_G = 16


def _sc_gather_body(ids_hbm, tok_tab, out_hbm, idx_v, rows_v, sem_g, sem_o,
                    *, seq, blocks_per_w):
    wid = lax.axis_index("s") * NUM_CORES + lax.axis_index("c")
    base_row = wid * blocks_per_w * SEQ_PAD
    n_chunks = blocks_per_w // IDX_CHUNK_BLKS

    def gather(i, buf):
        return pltpu.make_async_copy(
            tok_tab.at[idx_v.at[pl.ds(i * SEQ_PAD, SEQ_PAD)]],
            rows_v.at[buf], sem_g)

    def writeout(base, i, buf):
        return pltpu.make_async_copy(
            rows_v.at[buf], out_hbm.at[pl.ds(base + i * SEQ_PAD, SEQ_PAD)],
            sem_o)

    def chunk_body(c, carry):
        chunk_row = base_row + c * IDX_CHUNK_BLKS * SEQ_PAD
        pltpu.sync_copy(ids_hbm.at[pl.ds(chunk_row, IDX_CHUNK_BLKS * SEQ_PAD)],
                        idx_v)
        gather(0, 0).start()

        def body(i, carry2):
            buf = lax.rem(i, 2)
            nbuf = lax.rem(i + 1, 2)

            @pl.when(i >= 1)
            def _():
                writeout(chunk_row, i - 1, nbuf).wait()

            @pl.when(i + 1 < IDX_CHUNK_BLKS)
            def _():
                gather(i + 1, nbuf).start()

            gather(i, buf).wait()
            writeout(chunk_row, i, buf).start()
            return carry2

        lax.fori_loop(0, IDX_CHUNK_BLKS, body, 0)
        writeout(chunk_row, IDX_CHUNK_BLKS - 1,
                 lax.rem(IDX_CHUNK_BLKS - 1, 2)).wait()
        return carry

    lax.fori_loop(0, n_chunks, chunk_body, 0)


def _sc_gather(ids_pad_flat, token_table, n_rows, seq):
    blocks_per_w = (n_rows // SEQ_PAD) // NW
    mesh = plsc.VectorSubcoreMesh(core_axis_name="c", subcore_axis_name="s")
    run = pl.kernel(
        functools.partial(_sc_gather_body, seq=seq,
                          blocks_per_w=blocks_per_w),
        mesh=mesh,
        out_type=jax.ShapeDtypeStruct((n_rows, D), jnp.float32),
        scratch_types=[
            pltpu.VMEM((IDX_CHUNK_BLKS * SEQ_PAD,), jnp.int32),
            pltpu.VMEM((2, SEQ_PAD, D), jnp.float32),
            pltpu.SemaphoreType.DMA,
            pltpu.SemaphoreType.DMA,
        ],
    )
    return run(ids_pad_flat, token_table)


def _tc_body(rows_ref, pid_ref, ptab_ref, out_ref, *, seq):
    g = out_ref.shape[0]
    onehot = (pid_ref[...]
              == lax.broadcasted_iota(jnp.int32, (g * SEQ_PAD, seq), 1)
              ).astype(jnp.float32)
    pos = jax.lax.dot(onehot, ptab_ref[...],
                      preferred_element_type=jnp.float32)
    total = rows_ref[...] + pos
    for b in range(g):
        out_ref[b] = total[b * SEQ_PAD:b * SEQ_PAD + seq, :]


def _tc_addpos(tok_rows, pid_pad_flat, position_table, bsz, seq):
    grid = (bsz // TC_G,)
    return pl.pallas_call(
        functools.partial(_tc_body, seq=seq),
        grid=grid,
        in_specs=[
            pl.BlockSpec((TC_G * SEQ_PAD, D), lambda i: (i, 0)),
            pl.BlockSpec((TC_G * SEQ_PAD, 1), lambda i: (i, 0)),
            pl.BlockSpec((seq, D), lambda i: (0, 0)),
        ],
        out_specs=pl.BlockSpec((TC_G, seq, D), lambda i: (i, 0, 0)),
        out_shape=jax.ShapeDtypeStruct((bsz, seq, D), jnp.float32),
    )(tok_rows, pid_pad_flat, position_table)


def kernel(input_ids, position_ids, token_table, position_table):
    bsz, seq = input_ids.shape
    assert seq <= SEQ_PAD
    pad = ((0, 0), (0, SEQ_PAD - seq))
    vocab = token_table.shape[0]
    spread = (jnp.arange(bsz, dtype=jnp.int32)[:, None] * (SEQ_PAD - seq)
              + jnp.arange(SEQ_PAD - seq, dtype=jnp.int32)[None, :]) % vocab
    ids_pad = jnp.concatenate(
        [input_ids.astype(jnp.int32), spread], axis=1).reshape(bsz * SEQ_PAD)
    pid_pad = jnp.pad(position_ids.astype(jnp.int32),
                      pad).reshape(bsz * SEQ_PAD, 1)
    tok_rows = _sc_gather(ids_pad, token_table, bsz * SEQ_PAD, seq)
    return _tc_addpos(tok_rows, pid_pad, position_table, bsz, seq)
